# Initial kernel scaffold; baseline (speedup 1.0000x reference)
import jax
import jax.numpy as jnp
from jax.experimental import pallas as pl


def kernel(x, edge_index, W1, b1, Wmu, bmu, Wls, bls):
    raise NotImplementedError("write your pallas kernel here")


# speedup vs baseline: 14.5935x; 14.5935x over previous
import functools

import jax
import jax.numpy as jnp
from jax import lax
from jax.experimental import pallas as pl
from jax.experimental.pallas import tpu as pltpu
from jax.experimental.pallas import tpu_sc as plsc

_N = 10000
_C = 128
_NC = 2
_NS = 16
_NW = _NC * _NS
_RN = 640
_NP = _RN * _NS
_K = 80

_mesh = plsc.VectorSubcoreMesh(core_axis_name="c", subcore_axis_name="s")


def _sc_histogram(dst, ones_rows, zeros_rows):
    E = dst.shape[0]
    ew = E // _NW
    steps = ew // _K

    @functools.partial(
        pl.kernel,
        out_type=jax.ShapeDtypeStruct((_NC, _NP, _C), jnp.float32),
        mesh=_mesh,
        scratch_types=[
            pltpu.VMEM((1, _K), jnp.int32),
            pltpu.VMEM((_K, _C), jnp.float32),
            pltpu.VMEM_SHARED((_NP, _C), jnp.float32),
        ],
    )
    def hist(dst_hbm, ones_hbm, zeros_hbm, out_hbm, idx_v, ones_v, acc):
        c = lax.axis_index("c")
        s = lax.axis_index("s")
        wid = s * _NC + c
        pltpu.sync_copy(ones_hbm, ones_v)
        pltpu.sync_copy(zeros_hbm, acc.at[pl.ds(s * _RN, _RN)])
        plsc.subcore_barrier()

        @pl.loop(0, steps)
        def _(k):
            base = wid * ew + k * _K
            pltpu.sync_copy(dst_hbm.at[pl.ds(base, _K)], idx_v.at[0])
            pltpu.sync_copy(ones_v, acc.at[idx_v.at[0]], add=True)

        plsc.subcore_barrier()
        pltpu.sync_copy(acc.at[pl.ds(s * _RN, _RN)],
                        out_hbm.at[c].at[pl.ds(s * _RN, _RN)])

    return hist(dst, ones_rows, zeros_rows)


def _sc_aggregate(xs, src, dst, zeros_rows):
    E = src.shape[0]
    ew = E // _NW
    steps = ew // _K

    @functools.partial(
        pl.kernel,
        out_type=jax.ShapeDtypeStruct((_NC, _NP, _C), jnp.float32),
        mesh=_mesh,
        scratch_types=[
            pltpu.VMEM((1, _K), jnp.int32),
            pltpu.VMEM((1, _K), jnp.int32),
            pltpu.VMEM((_K, _C), jnp.float32),
            pltpu.VMEM_SHARED((_NP, _C), jnp.float32),
        ],
    )
    def agg(xs_hbm, src_hbm, dst_hbm, zeros_hbm, out_hbm,
            sidx_v, didx_v, rows_v, acc):
        c = lax.axis_index("c")
        s = lax.axis_index("s")
        wid = s * _NC + c
        pltpu.sync_copy(zeros_hbm, acc.at[pl.ds(s * _RN, _RN)])
        plsc.subcore_barrier()

        @pl.loop(0, steps)
        def _(k):
            base = wid * ew + k * _K
            pltpu.sync_copy(src_hbm.at[pl.ds(base, _K)], sidx_v.at[0])
            pltpu.sync_copy(dst_hbm.at[pl.ds(base, _K)], didx_v.at[0])
            pltpu.sync_copy(xs_hbm.at[sidx_v.at[0]], rows_v)
            pltpu.sync_copy(rows_v, acc.at[didx_v.at[0]], add=True)

        plsc.subcore_barrier()
        pltpu.sync_copy(acc.at[pl.ds(s * _RN, _RN)],
                        out_hbm.at[c].at[pl.ds(s * _RN, _RN)])

    return agg(xs, src, dst, zeros_rows)


_BR = 2000


def _tc_prescale_body(d0_ref, d1_ref, x_ref, dinv_ref, xs_ref):
    deg = d0_ref[:, 0:1] + d1_ref[:, 0:1] + 1.0
    dinv = lax.rsqrt(deg)
    dinv_b = jnp.broadcast_to(dinv, (d0_ref.shape[0], _C))
    dinv_ref[...] = dinv_b
    xs_ref[...] = dinv_b * x_ref[...]


def _tc_prescale(d0, d1, x):
    return pl.pallas_call(
        _tc_prescale_body,
        grid=(_N // _BR,),
        in_specs=[
            pl.BlockSpec((_BR, _C), lambda i: (i, 0)),
            pl.BlockSpec((_BR, _C), lambda i: (i, 0)),
            pl.BlockSpec((_BR, _C), lambda i: (i, 0)),
        ],
        out_specs=[
            pl.BlockSpec((_BR, _C), lambda i: (i, 0)),
            pl.BlockSpec((_BR, _C), lambda i: (i, 0)),
        ],
        out_shape=[
            jax.ShapeDtypeStruct((_N, _C), jnp.float32),
            jax.ShapeDtypeStruct((_N, _C), jnp.float32),
        ],
    )(d0, d1, x)


def _tc_layer1_body(p0_ref, p1_ref, x_ref, dinv_ref, w_ref, b_ref,
                    h_ref, hs_ref):
    dinv = dinv_ref[...]
    g = dinv * (p0_ref[...] + p1_ref[...]) + dinv * dinv * x_ref[...]
    h = jnp.dot(g, w_ref[...], preferred_element_type=jnp.float32)
    h = jnp.maximum(h + b_ref[...], 0.0)
    h_ref[...] = h
    hs_ref[...] = dinv * h


def _tc_layer1(p0, p1, x, dinv, W1, b1):
    return pl.pallas_call(
        _tc_layer1_body,
        grid=(_N // _BR,),
        in_specs=[
            pl.BlockSpec((_BR, _C), lambda i: (i, 0)),
            pl.BlockSpec((_BR, _C), lambda i: (i, 0)),
            pl.BlockSpec((_BR, _C), lambda i: (i, 0)),
            pl.BlockSpec((_BR, _C), lambda i: (i, 0)),
            pl.BlockSpec((_C, _C), lambda i: (0, 0)),
            pl.BlockSpec((1, _C), lambda i: (0, 0)),
        ],
        out_specs=[
            pl.BlockSpec((_BR, _C), lambda i: (i, 0)),
            pl.BlockSpec((_BR, _C), lambda i: (i, 0)),
        ],
        out_shape=[
            jax.ShapeDtypeStruct((_N, _C), jnp.float32),
            jax.ShapeDtypeStruct((_N, _C), jnp.float32),
        ],
    )(p0, p1, x, dinv, W1, b1)


def _tc_layer2_body(q0_ref, q1_ref, h_ref, dinv_ref, wm_ref, bm_ref,
                    wl_ref, bl_ref, mu_ref, ls_ref):
    dinv = dinv_ref[...]
    g = dinv * (q0_ref[...] + q1_ref[...]) + dinv * dinv * h_ref[...]
    mu_ref[...] = jnp.dot(g, wm_ref[...],
                          preferred_element_type=jnp.float32) + bm_ref[...]
    ls_ref[...] = jnp.dot(g, wl_ref[...],
                          preferred_element_type=jnp.float32) + bl_ref[...]


def _tc_layer2(q0, q1, h, dinv, Wmu, bmu, Wls, bls):
    return pl.pallas_call(
        _tc_layer2_body,
        grid=(_N // _BR,),
        in_specs=[
            pl.BlockSpec((_BR, _C), lambda i: (i, 0)),
            pl.BlockSpec((_BR, _C), lambda i: (i, 0)),
            pl.BlockSpec((_BR, _C), lambda i: (i, 0)),
            pl.BlockSpec((_BR, _C), lambda i: (i, 0)),
            pl.BlockSpec((_C, _C), lambda i: (0, 0)),
            pl.BlockSpec((1, _C), lambda i: (0, 0)),
            pl.BlockSpec((_C, _C), lambda i: (0, 0)),
            pl.BlockSpec((1, _C), lambda i: (0, 0)),
        ],
        out_specs=[
            pl.BlockSpec((_BR, _C), lambda i: (i, 0)),
            pl.BlockSpec((_BR, _C), lambda i: (i, 0)),
        ],
        out_shape=[
            jax.ShapeDtypeStruct((_N, _C), jnp.float32),
            jax.ShapeDtypeStruct((_N, _C), jnp.float32),
        ],
    )(q0, q1, h, dinv, Wmu, bmu, Wls, bls)


def kernel(x, edge_index, W1, b1, Wmu, bmu, Wls, bls):
    src = edge_index[0].astype(jnp.int32)
    dst = edge_index[1].astype(jnp.int32)

    ones_rows = jnp.ones((_K, _C), jnp.float32)
    zeros_rows = jnp.zeros((_RN, _C), jnp.float32)
    b1r = b1.reshape(1, _C)
    bmur = bmu.reshape(1, _C)
    blsr = bls.reshape(1, _C)

    degp = _sc_histogram(dst, ones_rows, zeros_rows)
    dinv, xs = _tc_prescale(degp[0, :_N], degp[1, :_N], x)
    p = _sc_aggregate(xs, src, dst, zeros_rows)
    h, hs = _tc_layer1(p[0, :_N], p[1, :_N], x, dinv, W1, b1r)
    q = _sc_aggregate(hs, src, dst, zeros_rows)
    mu, ls = _tc_layer2(q[0, :_N], q[1, :_N], h, dinv, Wmu, bmur, Wls, blsr)
    return (mu, ls)
